# baseline (speedup 1.0000x reference)
import functools

import jax
import jax.numpy as jnp
from jax import lax
from jax.experimental import pallas as pl
from jax.experimental.pallas import tpu as pltpu
from jax.experimental.pallas import tpu_sc as plsc

_LANES = 16
_A = (2.0 * 3.14159) ** 0.5
_L2E = 1.4426950408889634


def _sc_gather_xe(et_flat, x_flat, mul_flat, bias_flat):
    total = et_flat.shape[0]
    info = plsc.get_sparse_core_info()
    nw = info.num_cores * info.num_subcores
    chunk = total // nw
    tbl = mul_flat.shape[0]
    nc = info.num_cores
    mesh = plsc.VectorSubcoreMesh(core_axis_name="c", subcore_axis_name="s")

    @functools.partial(
        pl.kernel,
        mesh=mesh,
        out_type=jax.ShapeDtypeStruct((total,), jnp.float32),
        compiler_params=pltpu.CompilerParams(needs_layout_passes=False),
        scratch_types=[
            pltpu.VMEM((chunk,), jnp.int32),
            pltpu.VMEM((chunk,), jnp.float32),
            pltpu.VMEM((tbl,), jnp.float32),
            pltpu.VMEM((tbl,), jnp.float32),
            pltpu.VMEM((chunk,), jnp.float32),
        ],
    )
    def sc_kernel(et_hbm, x_hbm, mul_hbm, bias_hbm, out_hbm,
                  idx_v, x_v, mul_t, bias_t, xe_v):
        wid = lax.axis_index("s") * nc + lax.axis_index("c")
        base = wid * chunk
        pltpu.sync_copy(mul_hbm, mul_t)
        pltpu.sync_copy(bias_hbm, bias_t)
        pltpu.sync_copy(et_hbm.at[pl.ds(base, chunk)], idx_v)
        pltpu.sync_copy(x_hbm.at[pl.ds(base, chunk)], x_v)

        def body(i, carry):
            sl = pl.ds(i * _LANES, _LANES)
            idx = idx_v[sl]
            m = plsc.load_gather(mul_t, [idx])
            b = plsc.load_gather(bias_t, [idx])
            xe_v[sl] = m * x_v[sl] + b
            return carry

        lax.fori_loop(0, chunk // _LANES, body, 0)
        pltpu.sync_copy(xe_v, out_hbm.at[pl.ds(base, chunk)])

    return sc_kernel(et_flat, x_flat, mul_flat, bias_flat)


def _tc_expand_piece(prev, xe2, means, stds, rows, step_off, n_steps, total):
    k_dim = means.shape[-1]
    grp = xe2.shape[1]
    n_grp = rows // grp

    def body(*refs):
        if len(refs) == 5:
            xe_ref, m_ref, s_ref, o_ref = refs[1:]
        else:
            xe_ref, m_ref, s_ref, o_ref = refs
        std = jnp.abs(s_ref[...]) + 1e-05
        inv = 1.0 / std
        c2 = (-0.5 * _L2E) * inv * inv
        lc = -_L2E * jnp.log(_A * std)
        xt = xe_ref[...].T
        for g in range(n_grp):
            col = xt[:, g:g + 1]
            d = col - m_ref[...]
            o_ref[g * grp:(g + 1) * grp, :] = jnp.exp2(d * d * c2 + lc)

    data_specs = [
        pl.BlockSpec((n_grp, grp), lambda i: (i, 0)),
        pl.BlockSpec((1, k_dim), lambda i: (0, 0)),
        pl.BlockSpec((1, k_dim), lambda i: (0, 0)),
    ]
    if prev is None:
        in_specs, args, aliases = data_specs, (xe2, means, stds), {}
    else:
        in_specs = [pl.BlockSpec(memory_space=pl.ANY)] + data_specs
        args, aliases = (prev, xe2, means, stds), {0: 0}

    return pl.pallas_call(
        body,
        grid=(n_steps,),
        in_specs=in_specs,
        out_specs=pl.BlockSpec((rows, k_dim),
                               lambda i: (i + step_off, 0)),
        out_shape=jax.ShapeDtypeStruct((total, k_dim), jnp.float32),
        input_output_aliases=aliases,
    )(*args)


def kernel(x, edge_types, means, stds, mul_w, bias_w):
    b, n, m = x.shape
    k_dim = means.shape[-1]
    total = b * n * m
    n_pieces = 2
    rows = 32768
    piece = total // n_pieces
    steps_per_piece = piece // rows
    et = edge_types.reshape(n_pieces, piece).astype(jnp.int32)
    xf = x.reshape(n_pieces, piece).astype(jnp.float32)
    mul_flat = mul_w.reshape(-1).astype(jnp.float32)
    bias_flat = bias_w.reshape(-1).astype(jnp.float32)
    means32 = means.astype(jnp.float32)
    stds32 = stds.astype(jnp.float32)

    out = None
    for p in range(n_pieces):
        xe = _sc_gather_xe(et[p], xf[p], mul_flat, bias_flat)
        xe2 = xe.reshape(piece // 128, 128)
        out = _tc_expand_piece(out, xe2, means32, stds32,
                               rows, p * steps_per_piece, steps_per_piece,
                               total)
    return out.reshape(b, n, m, k_dim).astype(means.dtype)
